# Initial kernel scaffold; baseline (speedup 1.0000x reference)
import jax
import jax.numpy as jnp
from jax.experimental import pallas as pl


def kernel(x, edge_index, edge_attr, edge_weight, batch, bond_W, bond_b, eps, w1, b1, w2, b2, lstm_Wih, lstm_Whh, lstm_bih, lstm_bhh, fc1_W, fc1_b, fc4_W, fc4_b):
    raise NotImplementedError("write your pallas kernel here")


# speedup vs baseline: 2.0340x; 2.0340x over previous
import functools

import jax
import jax.numpy as jnp
from jax import lax
from jax.experimental import pallas as pl
from jax.experimental.pallas import tpu as pltpu
from jax.experimental.pallas import tpu_sc as plsc

N = 10000
E = 320000
D = 128
NUM_LAYERS = 3
G = 64
PROC_STEPS = 6

NC = 2
NS = 16
NW = NC * NS
E_PER_W = E // NW
CHUNK = 80
N_CHUNKS = E_PER_W // CHUNK
N_PAD = 10240
ROWS_PER_SUB = N_PAD // NS

_HIGH = lax.Precision.HIGHEST


def _sc_edge_body(h_hbm, src_hbm, dst_hbm, ee_hbm, ew_hbm, out_hbm,
                  srcA, dstA, ewA, eeA, rowsA,
                  srcB, dstB, ewB, eeB, rowsB,
                  agg_sh, semLA, semLB, semGA, semGB):
  cid = lax.axis_index("c")
  sid = lax.axis_index("s")
  wid = cid * NS + sid
  w_base = wid * E_PER_W

  def _zero_row(i, _):
    for j in range(D // 16):
      rowsA[i, pl.ds(j * 16, 16)] = jnp.zeros((16,), jnp.float32)
    return 0
  lax.fori_loop(0, CHUNK, _zero_row, 0)
  for t in range(ROWS_PER_SUB // CHUNK):
    pltpu.sync_copy(rowsA, agg_sh.at[pl.ds(sid * ROWS_PER_SUB + t * CHUNK, CHUNK)])
  plsc.subcore_barrier()

  def lin_descs(c, bufs, sem):
    src_v, dst_v, ew_v, ee_v = bufs
    base = w_base + c * CHUNK
    return [
        pltpu.make_async_copy(src_hbm.at[pl.ds(base, CHUNK)], src_v, sem),
        pltpu.make_async_copy(dst_hbm.at[pl.ds(base, CHUNK)], dst_v, sem),
        pltpu.make_async_copy(ew_hbm.at[pl.ds(base, CHUNK)], ew_v, sem),
        pltpu.make_async_copy(ee_hbm.at[pl.ds(base, CHUNK)], ee_v, sem),
    ]

  def issue_lin(c, bufs, sem):
    for d in lin_descs(c, bufs, sem):
      d.start()

  def wait_lin(c, bufs, sem):
    for d in lin_descs(c, bufs, sem):
      d.wait()

  def compute(ew_v, ee_v, rows_v):
    def _group(g, _):
      wv = ew_v[pl.ds(g * 16, 16)]

      def _feat(j, _):
        for t in range(16):
          e = g * 16 + t
          hv = rows_v[e, pl.ds(j * 16, 16)]
          ev = ee_v[e, pl.ds(j * 16, 16)]
          rows_v[e, pl.ds(j * 16, 16)] = jnp.maximum(hv + ev, 0.0) * wv[t]
        return 0
      lax.fori_loop(0, D // 16, _feat, 0)
      return 0
    lax.fori_loop(0, CHUNK // 16, _group, 0)

  bufsA = (srcA, dstA, ewA, eeA)
  bufsB = (srcB, dstB, ewB, eeB)

  issue_lin(0, bufsA, semLA)
  wait_lin(0, bufsA, semLA)
  pltpu.async_copy(h_hbm.at[srcA], rowsA, semGA)
  issue_lin(1, bufsB, semLB)

  def pair(j, _):
    a = 2 * j
    pltpu.make_async_copy(h_hbm.at[srcA], rowsA, semGA).wait()
    wait_lin(a + 1, bufsB, semLB)
    pltpu.async_copy(h_hbm.at[srcB], rowsB, semGB)
    compute(ewA, eeA, rowsA)
    pltpu.sync_copy(rowsA, agg_sh.at[dstA], add=True)
    issue_lin(a + 2, bufsA, semLA)
    pltpu.make_async_copy(h_hbm.at[srcB], rowsB, semGB).wait()
    wait_lin(a + 2, bufsA, semLA)
    pltpu.async_copy(h_hbm.at[srcA], rowsA, semGA)
    compute(ewB, eeB, rowsB)
    pltpu.sync_copy(rowsB, agg_sh.at[dstB], add=True)

    @pl.when(j < (N_CHUNKS - 1) // 2 - 1)
    def _():
      issue_lin(a + 3, bufsB, semLB)
    return 0
  lax.fori_loop(0, (N_CHUNKS - 1) // 2, pair, 0)

  pltpu.make_async_copy(h_hbm.at[srcA], rowsA, semGA).wait()
  compute(ewA, eeA, rowsA)
  pltpu.sync_copy(rowsA, agg_sh.at[dstA], add=True)

  plsc.subcore_barrier()
  pltpu.sync_copy(agg_sh.at[pl.ds(sid * ROWS_PER_SUB, ROWS_PER_SUB)],
                  out_hbm.at[cid, pl.ds(sid * ROWS_PER_SUB, ROWS_PER_SUB)])


@functools.cache
def _sc_edge_agg_fn():
  buf = lambda: [
      pltpu.VMEM((CHUNK,), jnp.int32),
      pltpu.VMEM((CHUNK,), jnp.int32),
      pltpu.VMEM((CHUNK,), jnp.float32),
      pltpu.VMEM((CHUNK, D), jnp.float32),
      pltpu.VMEM((CHUNK, D), jnp.float32),
  ]
  return pl.kernel(
      _sc_edge_body,
      mesh=plsc.VectorSubcoreMesh(core_axis_name="c", subcore_axis_name="s"),
      out_type=jax.ShapeDtypeStruct((NC, N_PAD, D), jnp.float32),
      scratch_types=[
          *buf(), *buf(),
          pltpu.VMEM_SHARED((N_PAD, D), jnp.float32),
          pltpu.SemaphoreType.DMA,
          pltpu.SemaphoreType.DMA,
          pltpu.SemaphoreType.DMA,
          pltpu.SemaphoreType.DMA,
      ],
  )


def _sc_edge_agg(h, src, dst, ee, ew):
  return _sc_edge_agg_fn()(h, src, dst, ee, ew)[:, :N, :]


_BE = 8000


def _ee_body(attr_ref, w_ref, b_ref, out_ref):
  a = attr_ref[...]
  for l in range(NUM_LAYERS):
    acc = jnp.broadcast_to(b_ref[l][None, :], (_BE, D))
    for k in range(4):
      acc = acc + a[:, k:k + 1] * w_ref[l, k][None, :]
    out_ref[l] = acc


def _edge_emb(edge_attr, bond_W, bond_b):
  return pl.pallas_call(
      _ee_body,
      grid=(E // _BE,),
      in_specs=[
          pl.BlockSpec((_BE, 4), lambda i: (i, 0)),
          pl.BlockSpec((NUM_LAYERS, 4, D), lambda i: (0, 0, 0)),
          pl.BlockSpec((NUM_LAYERS, D), lambda i: (0, 0)),
      ],
      out_specs=pl.BlockSpec((NUM_LAYERS, _BE, D), lambda i: (0, i, 0)),
      out_shape=jax.ShapeDtypeStruct((NUM_LAYERS, E, D), jnp.float32),
  )(edge_attr, bond_W, bond_b)


_BN = 2000


def _mlp_body(h_ref, p0_ref, p1_ref, eps_ref, w1_ref, b1_ref, w2_ref, b2_ref,
              out_ref):
  z = h_ref[...] * eps_ref[0, 0] + p0_ref[...] + p1_ref[...]
  z = jnp.maximum(jnp.dot(z, w1_ref[...], precision=_HIGH) + b1_ref[...], 0.0)
  z = jnp.dot(z, w2_ref[...], precision=_HIGH) + b2_ref[...]
  out_ref[...] = jnp.maximum(z, 0.0)


def _node_mlp(h, p0, p1, epsv, w1, b1, w2, b2):
  full = lambda shape: pl.BlockSpec(shape, lambda i: tuple(0 for _ in shape))
  return pl.pallas_call(
      _mlp_body,
      grid=(N // _BN,),
      in_specs=[
          pl.BlockSpec((_BN, D), lambda i: (i, 0)),
          pl.BlockSpec((_BN, D), lambda i: (i, 0)),
          pl.BlockSpec((_BN, D), lambda i: (i, 0)),
          full((1, 1)),
          full((D, D)),
          full((1, D)),
          full((D, D)),
          full((1, D)),
      ],
      out_specs=pl.BlockSpec((_BN, D), lambda i: (i, 0)),
      out_shape=jax.ShapeDtypeStruct((N, D), jnp.float32),
  )(h, p0, p1, epsv, w1, b1, w2, b2)


def _s2s_body(h_ref, batch_ref, wih_ref, whh_ref, bi_ref, bh_ref,
              fc1w_ref, fc1b_ref, fc4w_ref, fc4b_ref, out_ref):
  h = h_ref[...]
  bm = (batch_ref[...] ==
        lax.broadcasted_iota(jnp.int32, (N, G), 1)).astype(jnp.float32)
  wih = wih_ref[...]
  whh = whh_ref[...]
  bias = bi_ref[...] + bh_ref[...]

  q_star = jnp.zeros((G, 2 * D), jnp.float32)
  hh = jnp.zeros((G, D), jnp.float32)
  cc = jnp.zeros((G, D), jnp.float32)
  for _ in range(PROC_STEPS):
    gates = (jnp.dot(q_star, wih, precision=_HIGH) +
             jnp.dot(hh, whh, precision=_HIGH) + bias)
    i_ = jax.nn.sigmoid(gates[:, :D])
    f_ = jax.nn.sigmoid(gates[:, D:2 * D])
    g_ = jnp.tanh(gates[:, 2 * D:3 * D])
    o_ = jax.nn.sigmoid(gates[:, 3 * D:])
    cc = f_ * cc + i_ * g_
    hh = o_ * jnp.tanh(cc)
    q = hh

    qb = jnp.dot(bm, q, precision=_HIGH)
    e2 = jnp.sum(h * qb, axis=1, keepdims=True)
    masked = jnp.where(bm > 0, e2, -jnp.inf)
    em = jnp.max(masked, axis=0, keepdims=True)
    em = jnp.where(jnp.isfinite(em), em, 0.0)
    em_n = lax.dot_general(bm, em, (((1,), (1,)), ((), ())),
                           precision=_HIGH)
    a2 = jnp.exp(e2 - em_n)
    asum = lax.dot_general(bm, a2, (((0,), (0,)), ((), ())),
                           precision=_HIGH)
    asum_n = lax.dot_general(bm, asum, (((1,), (0,)), ((), ())),
                             precision=_HIGH)
    a2 = a2 / (asum_n + 1e-16)
    r = lax.dot_general(bm, a2 * h, (((0,), (0,)), ((), ())),
                        precision=_HIGH)
    q_star = jnp.concatenate([q, r], axis=1)

  o1 = jnp.maximum(jnp.dot(q_star, fc1w_ref[...], precision=_HIGH)
                   + fc1b_ref[...], 0.0)
  out_ref[...] = jnp.dot(o1, fc4w_ref[...], precision=_HIGH) + fc4b_ref[...]


def _set2set(h, batch2d, wih, whh, bih, bhh, fc1w, fc1b, fc4w, fc4b):
  nclass = fc4w.shape[1]
  return pl.pallas_call(
      _s2s_body,
      out_shape=jax.ShapeDtypeStruct((G, nclass), jnp.float32),
  )(h, batch2d, wih, whh, bih, bhh, fc1w, fc1b, fc4w, fc4b)


def kernel(x, edge_index, edge_attr, edge_weight, batch, bond_W, bond_b, eps,
           w1, b1, w2, b2, lstm_Wih, lstm_Whh, lstm_bih, lstm_bhh,
           fc1_W, fc1_b, fc4_W, fc4_b):
  src = edge_index[0]
  dst = edge_index[1]
  ee_all = _edge_emb(edge_attr, bond_W, bond_b)

  h = x
  for l in range(NUM_LAYERS):
    p = _sc_edge_agg(h, src, dst, ee_all[l], edge_weight)
    h = _node_mlp(h, p[0], p[1], (1.0 + eps[l]).reshape(1, 1),
                  w1[l], b1[l].reshape(1, D), w2[l], b2[l].reshape(1, D))

  return _set2set(h, batch.reshape(N, 1).astype(jnp.int32),
                  lstm_Wih, lstm_Whh, lstm_bih.reshape(1, -1),
                  lstm_bhh.reshape(1, -1), fc1_W, fc1_b.reshape(1, -1),
                  fc4_W, fc4_b.reshape(1, -1))


# speedup vs baseline: 2.1615x; 1.0627x over previous
import functools

import jax
import jax.numpy as jnp
from jax import lax
from jax.experimental import pallas as pl
from jax.experimental.pallas import tpu as pltpu
from jax.experimental.pallas import tpu_sc as plsc

N = 10000
E = 320000
D = 128
NUM_LAYERS = 3
G = 64
PROC_STEPS = 6

NC = 2
NS = 16
NW = NC * NS
E_PER_W = E // NW
CHUNK = 80
N_CHUNKS = E_PER_W // CHUNK
N_PAD = 10240
ROWS_PER_SUB = N_PAD // NS

_HIGH = lax.Precision.HIGHEST


ZROWS = 128


def _sc_edge_body(h_hbm, src_hbm, dst_hbm, ee_hbm, ew_hbm, out_hbm,
                  src_v, dst_v, rows_v, ee_v, ew_v, zbuf_v, agg_sh, sem):
  cid = lax.axis_index("c")
  sid = lax.axis_index("s")
  wid = cid * NS + sid

  def _zero_row(i, _):
    for j in range(D // 16):
      zbuf_v[i, pl.ds(j * 16, 16)] = jnp.zeros((16,), jnp.float32)
    return 0
  lax.fori_loop(0, ZROWS, _zero_row, 0)
  for t in range(ROWS_PER_SUB // ZROWS):
    pltpu.sync_copy(zbuf_v, agg_sh.at[pl.ds(sid * ROWS_PER_SUB + t * ZROWS, ZROWS)])
  plsc.subcore_barrier()

  def _chunk(i, _):
    base = wid * E_PER_W + i * CHUNK
    pltpu.sync_copy(src_hbm.at[pl.ds(base, CHUNK)], src_v)
    pltpu.sync_copy(dst_hbm.at[pl.ds(base, CHUNK)], dst_v)
    pltpu.sync_copy(ew_hbm.at[pl.ds(base, CHUNK)], ew_v)
    pltpu.sync_copy(ee_hbm.at[pl.ds(base, CHUNK)], ee_v)
    pltpu.async_copy(h_hbm.at[src_v], rows_v, sem).wait()

    def _group(g, _):
      wv = ew_v[pl.ds(g * 16, 16)]
      for t in range(16):
        w = wv[t]
        e = g * 16 + t
        for j in range(D // 16):
          hv = rows_v[e, pl.ds(j * 16, 16)]
          ev = ee_v[e, pl.ds(j * 16, 16)]
          rows_v[e, pl.ds(j * 16, 16)] = jnp.maximum(hv + ev, 0.0) * w
      return 0
    lax.fori_loop(0, CHUNK // 16, _group, 0)

    pltpu.sync_copy(rows_v, agg_sh.at[dst_v], add=True)
    return 0
  lax.fori_loop(0, N_CHUNKS, _chunk, 0)

  plsc.subcore_barrier()
  pltpu.sync_copy(agg_sh.at[pl.ds(sid * ROWS_PER_SUB, ROWS_PER_SUB)],
                  out_hbm.at[cid, pl.ds(sid * ROWS_PER_SUB, ROWS_PER_SUB)])


@functools.cache
def _sc_edge_agg_fn():
  return pl.kernel(
      _sc_edge_body,
      mesh=plsc.VectorSubcoreMesh(core_axis_name="c", subcore_axis_name="s"),
      out_type=jax.ShapeDtypeStruct((NC, N_PAD, D), jnp.float32),
      scratch_types=[
          pltpu.VMEM((CHUNK,), jnp.int32),
          pltpu.VMEM((CHUNK,), jnp.int32),
          pltpu.VMEM((CHUNK, D), jnp.float32),
          pltpu.VMEM((CHUNK, D), jnp.float32),
          pltpu.VMEM((CHUNK,), jnp.float32),
          pltpu.VMEM((ZROWS, D), jnp.float32),
          pltpu.VMEM_SHARED((N_PAD, D), jnp.float32),
          pltpu.SemaphoreType.DMA,
      ],
  )


def _sc_edge_agg(h, src, dst, ee, ew):
  return _sc_edge_agg_fn()(h, src, dst, ee, ew)[:, :N, :]


_BE = 8000


def _ee_body(attr_ref, w_ref, b_ref, out_ref):
  a = attr_ref[...]
  for l in range(NUM_LAYERS):
    out_ref[l] = jnp.dot(a, w_ref[l]) + b_ref[l][None, :]


def _edge_emb(edge_attr, bond_W, bond_b):
  return pl.pallas_call(
      _ee_body,
      grid=(E // _BE,),
      in_specs=[
          pl.BlockSpec((_BE, 4), lambda i: (i, 0)),
          pl.BlockSpec((NUM_LAYERS, 4, D), lambda i: (0, 0, 0)),
          pl.BlockSpec((NUM_LAYERS, D), lambda i: (0, 0)),
      ],
      out_specs=pl.BlockSpec((NUM_LAYERS, _BE, D), lambda i: (0, i, 0)),
      out_shape=jax.ShapeDtypeStruct((NUM_LAYERS, E, D), jnp.float32),
  )(edge_attr, bond_W, bond_b)


_BN = 2000


def _mlp_body(h_ref, p0_ref, p1_ref, eps_ref, w1_ref, b1_ref, w2_ref, b2_ref,
              out_ref):
  z = h_ref[...] * eps_ref[0, 0] + p0_ref[...] + p1_ref[...]
  z = jnp.maximum(jnp.dot(z, w1_ref[...]) + b1_ref[...], 0.0)
  z = jnp.dot(z, w2_ref[...]) + b2_ref[...]
  out_ref[...] = jnp.maximum(z, 0.0)


def _node_mlp(h, p0, p1, epsv, w1, b1, w2, b2):
  full = lambda shape: pl.BlockSpec(shape, lambda i: tuple(0 for _ in shape))
  return pl.pallas_call(
      _mlp_body,
      grid=(N // _BN,),
      in_specs=[
          pl.BlockSpec((_BN, D), lambda i: (i, 0)),
          pl.BlockSpec((_BN, D), lambda i: (i, 0)),
          pl.BlockSpec((_BN, D), lambda i: (i, 0)),
          full((1, 1)),
          full((D, D)),
          full((1, D)),
          full((D, D)),
          full((1, D)),
      ],
      out_specs=pl.BlockSpec((_BN, D), lambda i: (i, 0)),
      out_shape=jax.ShapeDtypeStruct((N, D), jnp.float32),
  )(h, p0, p1, epsv, w1, b1, w2, b2)


def _s2s_body(h_ref, batch_ref, wih_ref, whh_ref, bi_ref, bh_ref,
              fc1w_ref, fc1b_ref, fc4w_ref, fc4b_ref, out_ref):
  h = h_ref[...]
  bm = (batch_ref[...] ==
        lax.broadcasted_iota(jnp.int32, (N, G), 1)).astype(jnp.float32)
  wih = wih_ref[...]
  whh = whh_ref[...]
  bias = bi_ref[...] + bh_ref[...]

  q_star = jnp.zeros((G, 2 * D), jnp.float32)
  hh = jnp.zeros((G, D), jnp.float32)
  cc = jnp.zeros((G, D), jnp.float32)
  for _ in range(PROC_STEPS):
    gates = (jnp.dot(q_star, wih, precision=_HIGH) +
             jnp.dot(hh, whh, precision=_HIGH) + bias)
    i_ = jax.nn.sigmoid(gates[:, :D])
    f_ = jax.nn.sigmoid(gates[:, D:2 * D])
    g_ = jnp.tanh(gates[:, 2 * D:3 * D])
    o_ = jax.nn.sigmoid(gates[:, 3 * D:])
    cc = f_ * cc + i_ * g_
    hh = o_ * jnp.tanh(cc)
    q = hh

    qb = jnp.dot(bm, q, precision=_HIGH)
    e2 = jnp.sum(h * qb, axis=1, keepdims=True)
    masked = jnp.where(bm > 0, e2, -jnp.inf)
    em = jnp.max(masked, axis=0, keepdims=True)
    em = jnp.where(jnp.isfinite(em), em, 0.0)
    em_n = lax.dot_general(bm, em, (((1,), (1,)), ((), ())),
                           precision=_HIGH)
    a2 = jnp.exp(e2 - em_n)
    asum = lax.dot_general(bm, a2, (((0,), (0,)), ((), ())),
                           precision=_HIGH)
    asum_n = lax.dot_general(bm, asum, (((1,), (0,)), ((), ())),
                             precision=_HIGH)
    a2 = a2 / (asum_n + 1e-16)
    r = lax.dot_general(bm, a2 * h, (((0,), (0,)), ((), ())),
                        precision=_HIGH)
    q_star = jnp.concatenate([q, r], axis=1)

  o1 = jnp.maximum(jnp.dot(q_star, fc1w_ref[...], precision=_HIGH)
                   + fc1b_ref[...], 0.0)
  out_ref[...] = jnp.dot(o1, fc4w_ref[...], precision=_HIGH) + fc4b_ref[...]


def _set2set(h, batch2d, wih, whh, bih, bhh, fc1w, fc1b, fc4w, fc4b):
  nclass = fc4w.shape[1]
  return pl.pallas_call(
      _s2s_body,
      out_shape=jax.ShapeDtypeStruct((G, nclass), jnp.float32),
  )(h, batch2d, wih, whh, bih, bhh, fc1w, fc1b, fc4w, fc4b)


def kernel(x, edge_index, edge_attr, edge_weight, batch, bond_W, bond_b, eps,
           w1, b1, w2, b2, lstm_Wih, lstm_Whh, lstm_bih, lstm_bhh,
           fc1_W, fc1_b, fc4_W, fc4_b):
  src = edge_index[0]
  dst = edge_index[1]
  ee_all = _edge_emb(edge_attr, bond_W, bond_b)

  h = x
  for l in range(NUM_LAYERS):
    p = _sc_edge_agg(h, src, dst, ee_all[l], edge_weight)
    h = _node_mlp(h, p[0], p[1], (1.0 + eps[l]).reshape(1, 1),
                  w1[l], b1[l].reshape(1, D), w2[l], b2[l].reshape(1, D))

  return _set2set(h, batch.reshape(N, 1).astype(jnp.int32),
                  lstm_Wih, lstm_Whh, lstm_bih.reshape(1, -1),
                  lstm_bhh.reshape(1, -1), fc1_W, fc1_b.reshape(1, -1),
                  fc4_W, fc4_b.reshape(1, -1))


# speedup vs baseline: 2.7844x; 1.2882x over previous
import functools

import jax
import jax.numpy as jnp
from jax import lax
from jax.experimental import pallas as pl
from jax.experimental.pallas import tpu as pltpu
from jax.experimental.pallas import tpu_sc as plsc

N = 10000
E = 320000
D = 128
NUM_LAYERS = 3
G = 64
PROC_STEPS = 6

NC = 2
NS = 16
NW = NC * NS
E_PER_W = E // NW
CHUNK = 80
N_CHUNKS = E_PER_W // CHUNK
N_PAD = 10240
ROWS_PER_SUB = N_PAD // NS

_HIGH = lax.Precision.HIGHEST


ZROWS = 128


def _sc_edge_body(h_hbm, src_hbm, dst_hbm, ee_hbm, ew_hbm, out_hbm,
                  src_v, dst_v, rows_v, ee_v, ew_v, zbuf_v, agg_sh, sem):
  cid = lax.axis_index("c")
  sid = lax.axis_index("s")
  wid = cid * NS + sid

  def _zero_row(i, _):
    for j in range(D // 16):
      zbuf_v[i, pl.ds(j * 16, 16)] = jnp.zeros((16,), jnp.float32)
    return 0
  lax.fori_loop(0, ZROWS, _zero_row, 0)
  for t in range(ROWS_PER_SUB // ZROWS):
    pltpu.sync_copy(zbuf_v, agg_sh.at[pl.ds(sid * ROWS_PER_SUB + t * ZROWS, ZROWS)])
  plsc.subcore_barrier()

  def _chunk(i, _):
    base = wid * E_PER_W + i * CHUNK
    d_src = pltpu.make_async_copy(src_hbm.at[pl.ds(base, CHUNK)], src_v, sem)
    d_dst = pltpu.make_async_copy(dst_hbm.at[pl.ds(base, CHUNK)], dst_v, sem)
    d_ew = pltpu.make_async_copy(ew_hbm.at[pl.ds(base, CHUNK)], ew_v, sem)
    d_ee = pltpu.make_async_copy(ee_hbm.at[pl.ds(base, CHUNK)], ee_v, sem)
    for d in (d_src, d_dst, d_ew, d_ee):
      d.start()
    d_src.wait()
    d_gather = pltpu.async_copy(h_hbm.at[src_v], rows_v, sem)
    d_dst.wait()
    d_ew.wait()
    d_ee.wait()
    d_gather.wait()

    def _group(g, _):
      wv = ew_v[pl.ds(g * 16, 16)]
      for t in range(16):
        w = wv[t]
        e = g * 16 + t
        for j in range(D // 16):
          hv = rows_v[e, pl.ds(j * 16, 16)]
          ev = ee_v[e, pl.ds(j * 16, 16)]
          rows_v[e, pl.ds(j * 16, 16)] = jnp.maximum(hv + ev, 0.0) * w
      return 0
    lax.fori_loop(0, CHUNK // 16, _group, 0)

    pltpu.sync_copy(rows_v, agg_sh.at[dst_v], add=True)
    return 0
  lax.fori_loop(0, N_CHUNKS, _chunk, 0)

  plsc.subcore_barrier()
  pltpu.sync_copy(agg_sh.at[pl.ds(sid * ROWS_PER_SUB, ROWS_PER_SUB)],
                  out_hbm.at[cid, pl.ds(sid * ROWS_PER_SUB, ROWS_PER_SUB)])


@functools.cache
def _sc_edge_agg_fn():
  return pl.kernel(
      _sc_edge_body,
      mesh=plsc.VectorSubcoreMesh(core_axis_name="c", subcore_axis_name="s"),
      out_type=jax.ShapeDtypeStruct((NC, N_PAD, D), jnp.float32),
      scratch_types=[
          pltpu.VMEM((CHUNK,), jnp.int32),
          pltpu.VMEM((CHUNK,), jnp.int32),
          pltpu.VMEM((CHUNK, D), jnp.float32),
          pltpu.VMEM((CHUNK, D), jnp.float32),
          pltpu.VMEM((CHUNK,), jnp.float32),
          pltpu.VMEM((ZROWS, D), jnp.float32),
          pltpu.VMEM_SHARED((N_PAD, D), jnp.float32),
          pltpu.SemaphoreType.DMA,
      ],
  )


def _sc_edge_agg(h, src, dst, ee, ew):
  return _sc_edge_agg_fn()(h, src, dst, ee, ew)[:, :N, :]


_BE = 8000


def _ee_body(attr_ref, w_ref, b_ref, out_ref):
  a = attr_ref[...]
  for l in range(NUM_LAYERS):
    out_ref[l] = jnp.dot(a, w_ref[l]) + b_ref[l][None, :]


def _edge_emb(edge_attr, bond_W, bond_b):
  return pl.pallas_call(
      _ee_body,
      grid=(E // _BE,),
      in_specs=[
          pl.BlockSpec((_BE, 4), lambda i: (i, 0)),
          pl.BlockSpec((NUM_LAYERS, 4, D), lambda i: (0, 0, 0)),
          pl.BlockSpec((NUM_LAYERS, D), lambda i: (0, 0)),
      ],
      out_specs=pl.BlockSpec((NUM_LAYERS, _BE, D), lambda i: (0, i, 0)),
      out_shape=jax.ShapeDtypeStruct((NUM_LAYERS, E, D), jnp.float32),
  )(edge_attr, bond_W, bond_b)


_BN = 2000


def _mlp_body(h_ref, p0_ref, p1_ref, eps_ref, w1_ref, b1_ref, w2_ref, b2_ref,
              out_ref):
  z = h_ref[...] * eps_ref[0, 0] + p0_ref[...] + p1_ref[...]
  z = jnp.maximum(jnp.dot(z, w1_ref[...]) + b1_ref[...], 0.0)
  z = jnp.dot(z, w2_ref[...]) + b2_ref[...]
  out_ref[...] = jnp.maximum(z, 0.0)


def _node_mlp(h, p0, p1, epsv, w1, b1, w2, b2):
  full = lambda shape: pl.BlockSpec(shape, lambda i: tuple(0 for _ in shape))
  return pl.pallas_call(
      _mlp_body,
      grid=(N // _BN,),
      in_specs=[
          pl.BlockSpec((_BN, D), lambda i: (i, 0)),
          pl.BlockSpec((_BN, D), lambda i: (i, 0)),
          pl.BlockSpec((_BN, D), lambda i: (i, 0)),
          full((1, 1)),
          full((D, D)),
          full((1, D)),
          full((D, D)),
          full((1, D)),
      ],
      out_specs=pl.BlockSpec((_BN, D), lambda i: (i, 0)),
      out_shape=jax.ShapeDtypeStruct((N, D), jnp.float32),
  )(h, p0, p1, epsv, w1, b1, w2, b2)


def _s2s_body(h_ref, batch_ref, wih_ref, whh_ref, bi_ref, bh_ref,
              fc1w_ref, fc1b_ref, fc4w_ref, fc4b_ref, out_ref):
  h = h_ref[...]
  bm = (batch_ref[...] ==
        lax.broadcasted_iota(jnp.int32, (N, G), 1)).astype(jnp.float32)
  wih = wih_ref[...]
  whh = whh_ref[...]
  bias = bi_ref[...] + bh_ref[...]

  q_star = jnp.zeros((G, 2 * D), jnp.float32)
  hh = jnp.zeros((G, D), jnp.float32)
  cc = jnp.zeros((G, D), jnp.float32)
  for _ in range(PROC_STEPS):
    gates = (jnp.dot(q_star, wih, precision=_HIGH) +
             jnp.dot(hh, whh, precision=_HIGH) + bias)
    i_ = jax.nn.sigmoid(gates[:, :D])
    f_ = jax.nn.sigmoid(gates[:, D:2 * D])
    g_ = jnp.tanh(gates[:, 2 * D:3 * D])
    o_ = jax.nn.sigmoid(gates[:, 3 * D:])
    cc = f_ * cc + i_ * g_
    hh = o_ * jnp.tanh(cc)
    q = hh

    qb = jnp.dot(bm, q, precision=_HIGH)
    e2 = jnp.sum(h * qb, axis=1, keepdims=True)
    masked = jnp.where(bm > 0, e2, -jnp.inf)
    em = jnp.max(masked, axis=0, keepdims=True)
    em = jnp.where(jnp.isfinite(em), em, 0.0)
    em_n = lax.dot_general(bm, em, (((1,), (1,)), ((), ())),
                           precision=_HIGH)
    a2 = jnp.exp(e2 - em_n)
    asum = lax.dot_general(bm, a2, (((0,), (0,)), ((), ())),
                           precision=_HIGH)
    asum_n = lax.dot_general(bm, asum, (((1,), (0,)), ((), ())),
                             precision=_HIGH)
    a2 = a2 / (asum_n + 1e-16)
    r = lax.dot_general(bm, a2 * h, (((0,), (0,)), ((), ())),
                        precision=_HIGH)
    q_star = jnp.concatenate([q, r], axis=1)

  o1 = jnp.maximum(jnp.dot(q_star, fc1w_ref[...], precision=_HIGH)
                   + fc1b_ref[...], 0.0)
  out_ref[...] = jnp.dot(o1, fc4w_ref[...], precision=_HIGH) + fc4b_ref[...]


def _set2set(h, batch2d, wih, whh, bih, bhh, fc1w, fc1b, fc4w, fc4b):
  nclass = fc4w.shape[1]
  return pl.pallas_call(
      _s2s_body,
      out_shape=jax.ShapeDtypeStruct((G, nclass), jnp.float32),
  )(h, batch2d, wih, whh, bih, bhh, fc1w, fc1b, fc4w, fc4b)


def kernel(x, edge_index, edge_attr, edge_weight, batch, bond_W, bond_b, eps,
           w1, b1, w2, b2, lstm_Wih, lstm_Whh, lstm_bih, lstm_bhh,
           fc1_W, fc1_b, fc4_W, fc4_b):
  src = edge_index[0]
  dst = edge_index[1]
  ee_all = _edge_emb(edge_attr, bond_W, bond_b)

  h = x
  for l in range(NUM_LAYERS):
    p = _sc_edge_agg(h, src, dst, ee_all[l], edge_weight)
    h = _node_mlp(h, p[0], p[1], (1.0 + eps[l]).reshape(1, 1),
                  w1[l], b1[l].reshape(1, D), w2[l], b2[l].reshape(1, D))

  return _set2set(h, batch.reshape(N, 1).astype(jnp.int32),
                  lstm_Wih, lstm_Whh, lstm_bih.reshape(1, -1),
                  lstm_bhh.reshape(1, -1), fc1_W, fc1_b.reshape(1, -1),
                  fc4_W, fc4_b.reshape(1, -1))
